# Initial kernel scaffold; baseline (speedup 1.0000x reference)
import jax
import jax.numpy as jnp
from jax.experimental import pallas as pl


def kernel(feature, edge_index, W, b):
    raise NotImplementedError("write your pallas kernel here")


# speedup vs baseline: 2.1790x; 2.1790x over previous
import functools

import jax
import jax.numpy as jnp
from jax import lax
from jax.experimental import pallas as pl
from jax.experimental.pallas import tpu as pltpu
from jax.experimental.pallas import tpu_sc as plsc

N_NODES = 10000
N_FEATS = 128
ACC_ROWS = 10112
GARBAGE_ROW = 10000
NUM_CORES = 2
NUM_SUBCORES = 16
NW = NUM_CORES * NUM_SUBCORES
CHUNK = 128
BLK = 8
NBLK = 10
CHUNKS_PER_W = NBLK * BLK
ROWS_PER_TILE = ACC_ROWS // NUM_SUBCORES


def _sc_segment_sum(feature, src_c, dst_c):
    mesh = plsc.VectorSubcoreMesh(core_axis_name="c", subcore_axis_name="s")

    @functools.partial(
        pl.kernel,
        mesh=mesh,
        out_type=jax.ShapeDtypeStruct((NUM_CORES, ACC_ROWS, N_FEATS),
                                      jnp.float32),
        scratch_types=[
            pltpu.VMEM(((CHUNKS_PER_W + 2) * CHUNK,), jnp.int32),
            pltpu.VMEM((BLK, CHUNK), jnp.int32),
            pltpu.VMEM((CHUNK, N_FEATS), jnp.float32),
            pltpu.VMEM((CHUNK, N_FEATS), jnp.float32),
            pltpu.VMEM_SHARED((ACC_ROWS, N_FEATS), jnp.float32),
            pltpu.SemaphoreType.DMA,
            pltpu.SemaphoreType.DMA,
        ],
    )
    def seg_sum(feat_hbm, src_hbm, dst_hbm, out_hbm,
                src_v, dst_v, buf0, buf1, accum, sem0, sem1):
        cid = lax.axis_index("c")
        sid = lax.axis_index("s")
        wid = cid * NUM_SUBCORES + sid

        pltpu.sync_copy(src_hbm.at[wid], src_v)

        def zrow(r, carry):
            for j in range(N_FEATS // 16):
                buf0[r, pl.ds(j * 16, 16)] = jnp.zeros((16,), jnp.float32)
            return carry

        lax.fori_loop(0, CHUNK, zrow, 0)
        base = sid * ROWS_PER_TILE
        for k in range(ROWS_PER_TILE // CHUNK):
            pltpu.sync_copy(buf0, accum.at[pl.ds(base + k * CHUNK, CHUNK)])
        rem = ROWS_PER_TILE % CHUNK
        if rem:
            pltpu.sync_copy(
                buf0.at[pl.ds(0, rem)],
                accum.at[pl.ds(base + ROWS_PER_TILE - rem, rem)])
        plsc.subcore_barrier()

        def gather(g, buf, sem):
            idx = src_v.at[pl.ds(g * CHUNK, CHUNK)]
            pltpu.make_async_copy(feat_hbm.at[idx], buf, sem).start()

        def gwait(buf, sem):
            pltpu.make_async_copy(feat_hbm.at[pl.ds(0, CHUNK)], buf, sem).wait()

        gather(0, buf0, sem0)
        gather(1, buf1, sem1)

        def block(b, carry):
            pltpu.sync_copy(dst_hbm.at[wid, b], dst_v)
            for j in range(0, BLK, 2):
                g = b * BLK + j
                gwait(buf0, sem0)
                pltpu.sync_copy(buf0, accum.at[dst_v.at[j]], add=True)
                gather(g + 2, buf0, sem0)
                gwait(buf1, sem1)
                pltpu.sync_copy(buf1, accum.at[dst_v.at[j + 1]], add=True)
                gather(g + 3, buf1, sem1)
            return carry

        lax.fori_loop(0, NBLK, block, 0)
        gwait(buf0, sem0)
        gwait(buf1, sem1)

        plsc.subcore_barrier()
        pltpu.sync_copy(
            accum.at[pl.ds(base, ROWS_PER_TILE)],
            out_hbm.at[cid, pl.ds(base, ROWS_PER_TILE)])

    return seg_sum(feature, src_c, dst_c)


def _tc_linear_tanh(partials, W, b2):
    blk = ACC_ROWS // 8

    def body(p_ref, w_ref, b_ref, o_ref):
        s = p_ref[0] + p_ref[1]
        y = lax.dot_general(s, w_ref[...], (((1,), (1,)), ((), ())),
                            preferred_element_type=jnp.float32)
        o_ref[...] = jnp.tanh(y + b_ref[...])

    return pl.pallas_call(
        body,
        grid=(8,),
        in_specs=[
            pl.BlockSpec((NUM_CORES, blk, N_FEATS), lambda i: (0, i, 0)),
            pl.BlockSpec((N_FEATS, N_FEATS), lambda i: (0, 0)),
            pl.BlockSpec((1, N_FEATS), lambda i: (0, 0)),
        ],
        out_specs=pl.BlockSpec((blk, N_FEATS), lambda i: (i, 0)),
        out_shape=jax.ShapeDtypeStruct((ACC_ROWS, N_FEATS), jnp.float32),
    )(partials, W, b2)


def kernel(feature, edge_index, W, b):
    src = edge_index[0].astype(jnp.int32)
    dst = edge_index[1].astype(jnp.int32)
    n_edges = src.shape[0]
    total = NW * CHUNKS_PER_W * CHUNK
    pad = total - n_edges
    src_p = jnp.concatenate([src, jnp.zeros((pad,), jnp.int32)])
    dst_p = jnp.concatenate([dst, jnp.full((pad,), GARBAGE_ROW, jnp.int32)])
    src_c = src_p.reshape(NW, CHUNKS_PER_W * CHUNK)
    src_c = jnp.concatenate(
        [src_c, jnp.zeros((NW, 2 * CHUNK), jnp.int32)], axis=1)
    dst_c = dst_p.reshape(NW, NBLK, BLK, CHUNK)

    partials = _sc_segment_sum(feature, src_c, dst_c)
    out = _tc_linear_tanh(partials, W, b.reshape(1, N_FEATS))
    return out[:N_NODES]
